# Initial kernel scaffold; baseline (speedup 1.0000x reference)
import jax
import jax.numpy as jnp
from jax.experimental import pallas as pl


def kernel(x, edge_index, W1, a1_src, a1_dst, b1, W2, a2_src, a2_dst, b2):
    raise NotImplementedError("write your pallas kernel here")


# speedup vs baseline: 38.6305x; 38.6305x over previous
import functools
import jax
import jax.numpy as jnp
from jax import lax
from jax.experimental import pallas as pl
from jax.experimental.pallas import tpu as pltpu
from jax.experimental.pallas import tpu_sc as plsc

N = 10000
NPAD = 10240
E = 320000
NC = 2
NS = 16
NW = NC * NS
LANES = 16

F32 = jnp.float32
I32 = jnp.int32


def _make_edge_pass(n, e, feat, heads, ch):
    epw = e // NW
    chunks = epw // LANES
    rpt = NPAD // NS
    logc = ch.bit_length() - 1
    nparts = feat // LANES

    mesh = plsc.VectorSubcoreMesh(core_axis_name="c", subcore_axis_name="s")

    @functools.partial(
        pl.kernel,
        out_type=[
            jax.ShapeDtypeStruct((NC, NPAD, feat), F32),
            jax.ShapeDtypeStruct((NC, NPAD, 16), F32),
        ],
        mesh=mesh,
        scratch_types=[
            pltpu.VMEM((chunks, LANES), I32),
            pltpu.VMEM((chunks, LANES), I32),
            pltpu.VMEM((LANES, 16), F32),
            pltpu.VMEM((LANES, 16), F32),
            pltpu.VMEM((LANES, feat), F32),
            pltpu.VMEM((LANES, 16), F32),
            pltpu.VMEM((LANES, feat), F32),
            pltpu.VMEM_SHARED((NPAD, feat), F32),
            pltpu.VMEM_SHARED((NPAD, 16), F32),
            pltpu.SemaphoreType.DMA,
            pltpu.SemaphoreType.DMA,
            pltpu.SemaphoreType.DMA,
        ],
        compiler_params=pltpu.CompilerParams(use_tc_tiling_on_sc=False),
    )
    def edge_pass(al_hbm, h_hbm, src_hbm, dst_hbm, zacc_hbm, zden_hbm,
                  acc_out, den_out,
                  src_v, dst_v, sa, da, hh, tbuf, msg, acc_s, den_s,
                  sem_a, sem_b, sem_h):
        cid = lax.axis_index("c")
        sid = lax.axis_index("s")
        wid = cid * NS + sid

        iota = lax.iota(I32, LANES)
        rot8 = lax.bitwise_and(iota + 8, jnp.full((LANES,), 15, I32))
        lane_ok = iota < heads
        exp_idx = [
            lax.shift_right_logical(iota + LANES * j,
                                    jnp.full((LANES,), logc, I32))
            for j in range(nparts)
        ]

        gdn = lax.GatherDimensionNumbers(
            offset_dims=(), collapsed_slice_dims=(0,), start_index_map=(0,))

        def take(vec, idx):
            return lax.gather(
                vec, idx[:, None], dimension_numbers=gdn, slice_sizes=(1,),
                mode=lax.GatherScatterMode.PROMISE_IN_BOUNDS)

        row0 = sid * rpt
        pltpu.sync_copy(zacc_hbm, acc_s.at[pl.ds(row0, rpt)])
        pltpu.sync_copy(zden_hbm, den_s.at[pl.ds(row0, rpt)])
        pltpu.sync_copy(src_hbm.at[wid], src_v)
        pltpu.sync_copy(dst_hbm.at[wid], dst_v)
        plsc.subcore_barrier()

        def body(g, carry):
            src16 = src_v[g]
            dst16 = dst_v[g]
            d1 = pltpu.async_copy(al_hbm.at[src16], sa, sem_a)
            d2 = pltpu.async_copy(al_hbm.at[dst16], da, sem_b)
            d3 = pltpu.async_copy(h_hbm.at[src16], hh, sem_h)
            d1.wait()
            d2.wait()
            d3.wait()
            for i in range(LANES):
                a_vec = sa[i]
                b_vec = take(da[i], rot8)
                ee = a_vec + b_vec
                ee = jnp.maximum(ee, 0.2 * ee)
                tt = jnp.exp(ee)
                tt = jnp.where(lane_ok, tt, 0.0)
                tbuf[i] = tt
                for j in range(nparts):
                    te = take(tt, exp_idx[j])
                    msg[i, pl.ds(LANES * j, LANES)] = (
                        te * hh[i, pl.ds(LANES * j, LANES)])
            pltpu.sync_copy(tbuf, den_s.at[dst16], add=True)
            pltpu.sync_copy(msg, acc_s.at[dst16], add=True)
            return carry

        lax.fori_loop(0, chunks, body, 0)

        plsc.subcore_barrier()
        pltpu.sync_copy(acc_s.at[pl.ds(row0, rpt)],
                        acc_out.at[cid, pl.ds(row0, rpt)])
        pltpu.sync_copy(den_s.at[pl.ds(row0, rpt)],
                        den_out.at[cid, pl.ds(row0, rpt)])

    return edge_pass


_BLK = 1000
_GRID = N // _BLK


def _tc1_body(x_ref, w_ref, a_ref, h_ref, al_ref):
    h = jnp.dot(x_ref[...], w_ref[...], preferred_element_type=F32)
    h_ref[...] = h
    al_ref[...] = jnp.dot(h, a_ref[...], preferred_element_type=F32)


def _tc2_body(acc0, acc1, den0, den1, b_ref, w_ref, a_ref, p_ref,
              h_ref, al_ref):
    acc = acc0[...] + acc1[...]
    den = jnp.dot(den0[...] + den1[...], p_ref[...],
                  preferred_element_type=F32)
    x1 = acc / jnp.maximum(den, 1e-16) + b_ref[...]
    act = jnp.where(x1 > 0, x1, jnp.exp(x1) - 1.0)
    h2 = jnp.dot(act, w_ref[...], preferred_element_type=F32)
    h_ref[...] = h2
    al_ref[...] = jnp.dot(h2, a_ref[...], preferred_element_type=F32)


def _tc3_body(acc0, acc1, den0, den1, b_ref, p_ref, out_ref):
    acc = acc0[...] + acc1[...]
    den = jnp.dot(den0[...] + den1[...], p_ref[...],
                  preferred_element_type=F32)
    out_ref[...] = acc / jnp.maximum(den, 1e-16) + b_ref[...]


def _row_spec(cols):
    return pl.BlockSpec((_BLK, cols), lambda i: (i, 0))


def _full_spec(rows, cols):
    return pl.BlockSpec((rows, cols), lambda i: (0, 0))


def kernel(x, edge_index, W1, a1_src, a1_dst, b1, W2, a2_src, a2_dst, b2):
    epw = E // NW
    src = edge_index[0].reshape(NW, epw // LANES, LANES)
    dst = edge_index[1].reshape(NW, epw // LANES, LANES)

    def pack_a(a_s, a_d, din):
        h_, c_ = a_s.shape
        cols = []
        for k in range(16):
            if k < h_:
                col = jnp.zeros((din,), F32).at[k * c_:(k + 1) * c_].set(a_s[k])
            elif 8 <= k < 8 + h_:
                hh = k - 8
                col = jnp.zeros((din,), F32).at[hh * c_:(hh + 1) * c_].set(a_d[hh])
            else:
                col = jnp.zeros((din,), F32)
            cols.append(col)
        return jnp.stack(cols, axis=1)

    A1 = pack_a(a1_src, a1_dst, 64)
    A2 = pack_a(a2_src, a2_dst, 128)
    P1 = jnp.concatenate(
        [jnp.kron(jnp.eye(8, dtype=F32), jnp.ones((1, 8), F32)),
         jnp.zeros((8, 64), F32)], axis=0)
    P2 = jnp.concatenate(
        [jnp.ones((1, 128), F32), jnp.zeros((15, 128), F32)], axis=0)

    rpt = NPAD // NS
    z64 = jnp.zeros((rpt, 64), F32)
    z128 = jnp.zeros((rpt, 128), F32)
    z16 = jnp.zeros((rpt, 16), F32)

    h1, al1 = pl.pallas_call(
        _tc1_body,
        grid=(_GRID,),
        in_specs=[_row_spec(128), _full_spec(128, 64), _full_spec(64, 16)],
        out_specs=[_row_spec(64), _row_spec(16)],
        out_shape=[jax.ShapeDtypeStruct((N, 64), F32),
                   jax.ShapeDtypeStruct((N, 16), F32)],
    )(x, W1, A1)

    acc1, den1 = _make_edge_pass(N, E, 64, 8, 8)(al1, h1, src, dst, z64, z16)
    acc1 = acc1[:, :N]
    den1 = den1[:, :N]

    h2, al2 = pl.pallas_call(
        _tc2_body,
        grid=(_GRID,),
        in_specs=[_row_spec(64), _row_spec(64), _row_spec(16), _row_spec(16),
                  _full_spec(1, 64), _full_spec(64, 128), _full_spec(128, 16),
                  _full_spec(16, 64)],
        out_specs=[_row_spec(128), _row_spec(16)],
        out_shape=[jax.ShapeDtypeStruct((N, 128), F32),
                   jax.ShapeDtypeStruct((N, 16), F32)],
    )(acc1[0], acc1[1], den1[0], den1[1], b1.reshape(1, 64), W2, A2, P1)

    acc2, den2 = _make_edge_pass(N, E, 128, 1, 128)(al2, h2, src, dst,
                                                    z128, z16)
    acc2 = acc2[:, :N]
    den2 = den2[:, :N]

    out = pl.pallas_call(
        _tc3_body,
        grid=(_GRID,),
        in_specs=[_row_spec(128), _row_spec(128), _row_spec(16), _row_spec(16),
                  _full_spec(1, 128), _full_spec(16, 128)],
        out_specs=_row_spec(128),
        out_shape=jax.ShapeDtypeStruct((N, 128), F32),
    )(acc2[0], acc2[1], den2[0], den2[1], b2.reshape(1, 128), P2)

    return out
